# baseline (speedup 1.0000x reference)
import functools

import jax
import jax.numpy as jnp
from jax import lax
from jax.experimental import pallas as pl
from jax.experimental.pallas import tpu as pltpu, tpu_sc as plsc

INPUT_DIM = 26
NUM_EMB = 100000
OUT_DIM = 32
LANES = 16
PAD_W = 128

ROWS_PER_CHUNK = 64
LOOKUPS = ROWS_PER_CHUNK * INPUT_DIM

NC, NS = 2, 16
NW = NC * NS

P_TOT = INPUT_DIM * NUM_EMB * OUT_DIM // PAD_W
PW_STRIDE = P_TOT // NW // 8 * 8
P_CHUNK = 64
P_CHUNKS = -(-(P_TOT - 31 * PW_STRIDE) // P_CHUNK)
P_CHUNKS += P_CHUNKS % 2


def _mesh():
    return plsc.VectorSubcoreMesh(core_axis_name="c", subcore_axis_name="s")


def _build_depad():
    assert P_CHUNKS % 2 == 0

    @functools.partial(
        pl.kernel,
        mesh=_mesh(),
        out_type=jax.ShapeDtypeStruct((P_TOT, PAD_W), jnp.float32),
        scratch_types=[
            pltpu.VMEM((4 * P_CHUNK, OUT_DIM), jnp.float32),
            pltpu.VMEM((4 * P_CHUNK, OUT_DIM), jnp.float32),
            pltpu.VMEM((P_CHUNK, PAD_W), jnp.float32),
            pltpu.VMEM((P_CHUNK, PAD_W), jnp.float32),
            pltpu.SemaphoreType.DMA,
            pltpu.SemaphoreType.DMA,
            pltpu.SemaphoreType.DMA,
            pltpu.SemaphoreType.DMA,
        ],
    )
    def k1(tab_hbm, tabp_hbm, vin_a, vin_b, vout_a, vout_b,
           sem_ra, sem_rb, sem_wa, sem_wb):
        wid = lax.axis_index("s") * NC + lax.axis_index("c")
        start = wid * PW_STRIDE
        end = jnp.minimum(start + (PW_STRIDE + 16), P_TOT)

        def p_at(g):
            return pl.multiple_of(
                jnp.minimum(start + g * P_CHUNK, end - P_CHUNK), 8)

        def start_read(g, vin, sem):
            return pltpu.async_copy(
                tab_hbm.at[pl.ds(p_at(g) * 4, 4 * P_CHUNK), :], vin, sem)

        def wait_read(vin, sem):
            pltpu.make_async_copy(
                tab_hbm.at[pl.ds(0, 4 * P_CHUNK), :], vin, sem).wait()

        def drain_write(vout, sem):
            pltpu.make_async_copy(vout, tabp_hbm.at[pl.ds(0, P_CHUNK)],
                                  sem).wait()

        def shuffle_write(g, vin, vout, sem):
            for p in range(P_CHUNK):
                for q in range(4):
                    for j in range(2):
                        vout[p, pl.ds(q * OUT_DIM + j * LANES, LANES)] = (
                            vin[4 * p + q, pl.ds(j * LANES, LANES)]
                        )
            pltpu.async_copy(vout, tabp_hbm.at[pl.ds(p_at(g), P_CHUNK)], sem)

        start_read(0, vin_a, sem_ra)

        def pair_body(t, _):
            g = 2 * t
            start_read(g + 1, vin_b, sem_rb)
            wait_read(vin_a, sem_ra)

            @pl.when(t > 0)
            def _():
                drain_write(vout_a, sem_wa)

            shuffle_write(g, vin_a, vout_a, sem_wa)

            @pl.when(g + 2 < P_CHUNKS)
            def _():
                start_read(g + 2, vin_a, sem_ra)

            wait_read(vin_b, sem_rb)

            @pl.when(t > 0)
            def _():
                drain_write(vout_b, sem_wb)

            shuffle_write(g + 1, vin_b, vout_b, sem_wb)
            return 0

        lax.fori_loop(0, P_CHUNKS // 2, pair_body, 0)
        drain_write(vout_a, sem_wa)
        drain_write(vout_b, sem_wb)

    return k1


def _build_lookup(num_rows):
    rows_per_w = num_rows // NW
    chunks = rows_per_w // ROWS_PER_CHUNK
    assert chunks % 2 == 1

    @functools.partial(
        pl.kernel,
        mesh=_mesh(),
        compiler_params=pltpu.CompilerParams(use_tc_tiling_on_sc=False),
        out_type=jax.ShapeDtypeStruct((num_rows, OUT_DIM), jnp.float32),
        scratch_types=[
            pltpu.VMEM((LOOKUPS,), jnp.int32),
            pltpu.VMEM((LOOKUPS,), jnp.int32),
            pltpu.VMEM((LOOKUPS,), jnp.int32),
            pltpu.VMEM((LOOKUPS, OUT_DIM), jnp.float32),
            pltpu.VMEM((LOOKUPS, OUT_DIM), jnp.float32),
            pltpu.VMEM((ROWS_PER_CHUNK, OUT_DIM), jnp.float32),
            pltpu.SemaphoreType.DMA,
            pltpu.SemaphoreType.DMA,
        ],
    )
    def k2(tab_hbm, idx_hbm, offs_hbm, out_hbm,
           idx_a, idx_b, offs_v, rows_a, rows_b, out_v, sem_a, sem_b):
        wid = lax.axis_index("s") * NC + lax.axis_index("c")
        pltpu.sync_copy(offs_hbm, offs_v)
        idx_base = wid * (rows_per_w * INPUT_DIM)

        def stage(g, idx_v, rows_v, sem):
            off = pl.multiple_of(idx_base + g * LOOKUPS, 8)
            pltpu.sync_copy(idx_hbm.at[pl.ds(off, LOOKUPS)], idx_v)

            def offs_body(j, _):
                for kk in range(128 // LANES):
                    sl = pl.ds(j * 128 + kk * LANES, LANES)
                    idx_v[sl] = idx_v[sl] + offs_v[sl]
                return 0

            lax.fori_loop(0, LOOKUPS // 128, offs_body, 0)
            return pltpu.async_copy(tab_hbm.at[idx_v], rows_v, sem)

        def process(g, rows_v):
            def acc_body(c, _):
                base = c * INPUT_DIM
                a0 = rows_v[base, pl.ds(0, LANES)]
                a1 = rows_v[base, pl.ds(LANES, LANES)]
                for i in range(1, INPUT_DIM):
                    a0 = a0 + rows_v[base + i, pl.ds(0, LANES)]
                    a1 = a1 + rows_v[base + i, pl.ds(LANES, LANES)]
                out_v[c, pl.ds(0, LANES)] = a0
                out_v[c, pl.ds(LANES, LANES)] = a1
                return 0

            lax.fori_loop(0, ROWS_PER_CHUNK, acc_body, 0)
            pltpu.sync_copy(
                out_v,
                out_hbm.at[pl.ds(wid * rows_per_w + g * ROWS_PER_CHUNK,
                                 ROWS_PER_CHUNK)],
            )

        stage(0, idx_a, rows_a, sem_a)

        def pair_body(t, _):
            g = 2 * t
            cp_b = stage(g + 1, idx_b, rows_b, sem_b)
            pltpu.make_async_copy(tab_hbm.at[idx_a], rows_a, sem_a).wait()
            process(g, rows_a)
            stage(g + 2, idx_a, rows_a, sem_a)
            cp_b.wait()
            process(g + 1, rows_b)
            return 0

        lax.fori_loop(0, chunks // 2, pair_body, 0)
        pltpu.make_async_copy(tab_hbm.at[idx_a], rows_a, sem_a).wait()
        process(chunks - 1, rows_a)

    return k2


def kernel(int_vec, tables):
    bs, num_nodes, input_dim = int_vec.shape
    num_rows = bs * num_nodes
    n_flat = input_dim * tables.shape[1]
    tabp = lax.optimization_barrier(
        tables.reshape(n_flat * tables.shape[2] // PAD_W, PAD_W))
    tab_sc = tabp.reshape(n_flat, tables.shape[2])
    idx_flat = int_vec.reshape(num_rows * input_dim)
    offs = jnp.tile(
        jnp.arange(INPUT_DIM, dtype=jnp.int32) * NUM_EMB, ROWS_PER_CHUNK
    )
    out = _build_lookup(num_rows)(tab_sc, idx_flat, offs)
    return out.reshape(bs, num_nodes, tables.shape[2])


# speedup vs baseline: 1.2792x; 1.2792x over previous
import functools

import jax
import jax.numpy as jnp
from jax import lax
from jax.experimental import pallas as pl
from jax.experimental.pallas import tpu as pltpu, tpu_sc as plsc

INPUT_DIM = 26
NUM_EMB = 100000
OUT_DIM = 32
LANES = 16
PAD_W = 128

ROWS_PER_CHUNK = 64
LOOKUPS = ROWS_PER_CHUNK * INPUT_DIM

NC, NS = 2, 16
NW = NC * NS

P_TOT = INPUT_DIM * NUM_EMB * OUT_DIM // PAD_W
PW_STRIDE = P_TOT // NW // 8 * 8
P_CHUNK = 64
P_CHUNKS = -(-(P_TOT - 31 * PW_STRIDE) // P_CHUNK)
P_CHUNKS += -P_CHUNKS % 3


def _mesh():
    return plsc.VectorSubcoreMesh(core_axis_name="c", subcore_axis_name="s")


def _build_depad():
    assert P_CHUNKS % 3 == 0

    @functools.partial(
        pl.kernel,
        mesh=_mesh(),
        out_type=jax.ShapeDtypeStruct((P_TOT, PAD_W), jnp.float32),
        scratch_types=[
            pltpu.VMEM((4 * P_CHUNK, OUT_DIM), jnp.float32),
            pltpu.VMEM((4 * P_CHUNK, OUT_DIM), jnp.float32),
            pltpu.VMEM((4 * P_CHUNK, OUT_DIM), jnp.float32),
            pltpu.VMEM((P_CHUNK, PAD_W), jnp.float32),
            pltpu.VMEM((P_CHUNK, PAD_W), jnp.float32),
            pltpu.VMEM((P_CHUNK, PAD_W), jnp.float32),
            pltpu.SemaphoreType.DMA,
            pltpu.SemaphoreType.DMA,
            pltpu.SemaphoreType.DMA,
            pltpu.SemaphoreType.DMA,
            pltpu.SemaphoreType.DMA,
            pltpu.SemaphoreType.DMA,
        ],
    )
    def k1(tab_hbm, tabp_hbm, vin_a, vin_b, vin_c, vout_a, vout_b, vout_c,
           sem_ra, sem_rb, sem_rc, sem_wa, sem_wb, sem_wc):
        wid = lax.axis_index("s") * NC + lax.axis_index("c")
        start = wid * PW_STRIDE
        end = jnp.minimum(start + (PW_STRIDE + 16), P_TOT)

        def p_at(g):
            return pl.multiple_of(
                jnp.minimum(start + g * P_CHUNK, end - P_CHUNK), 8)

        def start_read(g, vin, sem):
            return pltpu.async_copy(
                tab_hbm.at[pl.ds(p_at(g) * 4, 4 * P_CHUNK), :], vin, sem)

        def wait_read(vin, sem):
            pltpu.make_async_copy(
                tab_hbm.at[pl.ds(0, 4 * P_CHUNK), :], vin, sem).wait()

        def drain_write(vout, sem):
            pltpu.make_async_copy(vout, tabp_hbm.at[pl.ds(0, P_CHUNK)],
                                  sem).wait()

        def shuffle_write(g, vin, vout, sem):
            for p in range(P_CHUNK):
                for q in range(4):
                    for j in range(2):
                        vout[p, pl.ds(q * OUT_DIM + j * LANES, LANES)] = (
                            vin[4 * p + q, pl.ds(j * LANES, LANES)]
                        )
            pltpu.async_copy(vout, tabp_hbm.at[pl.ds(p_at(g), P_CHUNK)], sem)

        start_read(0, vin_a, sem_ra)
        start_read(1, vin_b, sem_rb)
        start_read(2, vin_c, sem_rc)

        bufs = ((vin_a, sem_ra, vout_a, sem_wa),
                (vin_b, sem_rb, vout_b, sem_wb),
                (vin_c, sem_rc, vout_c, sem_wc))

        def triple_body(t, _):
            g = 3 * t
            for i, (vin, sem_r, vout, sem_w) in enumerate(bufs):
                wait_read(vin, sem_r)

                @pl.when(t > 0)
                def _():
                    drain_write(vout, sem_w)

                shuffle_write(g + i, vin, vout, sem_w)

                @pl.when(g + i + 3 < P_CHUNKS)
                def _():
                    start_read(g + i + 3, vin, sem_r)
            return 0

        lax.fori_loop(0, P_CHUNKS // 3, triple_body, 0)
        for _, _, vout, sem_w in bufs:
            drain_write(vout, sem_w)

    return k1


def _build_lookup(num_rows):
    rows_per_w = num_rows // NW
    chunks = rows_per_w // ROWS_PER_CHUNK
    assert chunks % 2 == 1

    @functools.partial(
        pl.kernel,
        mesh=_mesh(),
        compiler_params=pltpu.CompilerParams(use_tc_tiling_on_sc=False),
        out_type=jax.ShapeDtypeStruct((num_rows, OUT_DIM), jnp.float32),
        scratch_types=[
            pltpu.VMEM((LOOKUPS,), jnp.int32),
            pltpu.VMEM((LOOKUPS,), jnp.int32),
            pltpu.VMEM((LOOKUPS,), jnp.int32),
            pltpu.VMEM((LOOKUPS, OUT_DIM), jnp.float32),
            pltpu.VMEM((LOOKUPS, OUT_DIM), jnp.float32),
            pltpu.VMEM((ROWS_PER_CHUNK, OUT_DIM), jnp.float32),
            pltpu.SemaphoreType.DMA,
            pltpu.SemaphoreType.DMA,
        ],
    )
    def k2(tab_hbm, idx_hbm, offs_hbm, out_hbm,
           idx_a, idx_b, offs_v, rows_a, rows_b, out_v, sem_a, sem_b):
        wid = lax.axis_index("s") * NC + lax.axis_index("c")
        pltpu.sync_copy(offs_hbm, offs_v)
        idx_base = wid * (rows_per_w * INPUT_DIM)

        def stage(g, idx_v, rows_v, sem):
            off = pl.multiple_of(idx_base + g * LOOKUPS, 8)
            pltpu.sync_copy(idx_hbm.at[pl.ds(off, LOOKUPS)], idx_v)

            def offs_body(j, _):
                for kk in range(128 // LANES):
                    sl = pl.ds(j * 128 + kk * LANES, LANES)
                    idx_v[sl] = idx_v[sl] + offs_v[sl]
                return 0

            lax.fori_loop(0, LOOKUPS // 128, offs_body, 0)
            return pltpu.async_copy(tab_hbm.at[idx_v], rows_v, sem)

        def process(g, rows_v):
            def acc_body(c, _):
                base = c * INPUT_DIM
                a0 = rows_v[base, pl.ds(0, LANES)]
                a1 = rows_v[base, pl.ds(LANES, LANES)]
                for i in range(1, INPUT_DIM):
                    a0 = a0 + rows_v[base + i, pl.ds(0, LANES)]
                    a1 = a1 + rows_v[base + i, pl.ds(LANES, LANES)]
                out_v[c, pl.ds(0, LANES)] = a0
                out_v[c, pl.ds(LANES, LANES)] = a1
                return 0

            lax.fori_loop(0, ROWS_PER_CHUNK, acc_body, 0)
            pltpu.sync_copy(
                out_v,
                out_hbm.at[pl.ds(wid * rows_per_w + g * ROWS_PER_CHUNK,
                                 ROWS_PER_CHUNK)],
            )

        stage(0, idx_a, rows_a, sem_a)

        def pair_body(t, _):
            g = 2 * t
            cp_b = stage(g + 1, idx_b, rows_b, sem_b)
            pltpu.make_async_copy(tab_hbm.at[idx_a], rows_a, sem_a).wait()
            process(g, rows_a)
            stage(g + 2, idx_a, rows_a, sem_a)
            cp_b.wait()
            process(g + 1, rows_b)
            return 0

        lax.fori_loop(0, chunks // 2, pair_body, 0)
        pltpu.make_async_copy(tab_hbm.at[idx_a], rows_a, sem_a).wait()
        process(chunks - 1, rows_a)

    return k2


def kernel(int_vec, tables):
    bs, num_nodes, input_dim = int_vec.shape
    num_rows = bs * num_nodes
    tab2d = tables.reshape(input_dim * tables.shape[1], tables.shape[2])
    tabp = _build_depad()(tab2d)
    tab_sc = tabp.reshape(input_dim * tables.shape[1], tables.shape[2])
    idx_flat = int_vec.reshape(num_rows * input_dim)
    offs = jnp.tile(
        jnp.arange(INPUT_DIM, dtype=jnp.int32) * NUM_EMB, ROWS_PER_CHUNK
    )
    out = _build_lookup(num_rows)(tab_sc, idx_flat, offs)
    return out.reshape(bs, num_nodes, tables.shape[2])


# speedup vs baseline: 1.3207x; 1.0324x over previous
import functools

import jax
import jax.numpy as jnp
from jax import lax
from jax.experimental import pallas as pl
from jax.experimental.pallas import tpu as pltpu, tpu_sc as plsc

INPUT_DIM = 26
NUM_EMB = 100000
OUT_DIM = 32
LANES = 16
PAD_W = 128

ROWS_PER_CHUNK = 64
LOOKUPS = ROWS_PER_CHUNK * INPUT_DIM

NC, NS = 2, 16
NW = NC * NS

P_TOT = INPUT_DIM * NUM_EMB * OUT_DIM // PAD_W
PW_STRIDE = P_TOT // NW // 8 * 8
P_CHUNK = 64
P_CHUNKS = -(-(P_TOT - 31 * PW_STRIDE) // P_CHUNK)
P_CHUNKS += -P_CHUNKS % 3


def _mesh():
    return plsc.VectorSubcoreMesh(core_axis_name="c", subcore_axis_name="s")


def _build_depad():
    assert P_CHUNKS % 3 == 0

    @functools.partial(
        pl.kernel,
        mesh=_mesh(),
        out_type=jax.ShapeDtypeStruct((P_TOT, PAD_W), jnp.float32),
        scratch_types=[
            pltpu.VMEM((4 * P_CHUNK, OUT_DIM), jnp.float32),
            pltpu.VMEM((4 * P_CHUNK, OUT_DIM), jnp.float32),
            pltpu.VMEM((4 * P_CHUNK, OUT_DIM), jnp.float32),
            pltpu.VMEM((P_CHUNK, PAD_W), jnp.float32),
            pltpu.VMEM((P_CHUNK, PAD_W), jnp.float32),
            pltpu.VMEM((P_CHUNK, PAD_W), jnp.float32),
            pltpu.SemaphoreType.DMA,
            pltpu.SemaphoreType.DMA,
            pltpu.SemaphoreType.DMA,
            pltpu.SemaphoreType.DMA,
            pltpu.SemaphoreType.DMA,
            pltpu.SemaphoreType.DMA,
        ],
    )
    def k1(tab_hbm, tabp_hbm, vin_a, vin_b, vin_c, vout_a, vout_b, vout_c,
           sem_ra, sem_rb, sem_rc, sem_wa, sem_wb, sem_wc):
        wid = lax.axis_index("s") * NC + lax.axis_index("c")
        start = wid * PW_STRIDE
        end = jnp.minimum(start + (PW_STRIDE + 16), P_TOT)

        def p_at(g):
            return pl.multiple_of(
                jnp.minimum(start + g * P_CHUNK, end - P_CHUNK), 8)

        def start_read(g, vin, sem):
            return pltpu.async_copy(
                tab_hbm.at[pl.ds(p_at(g) * 4, 4 * P_CHUNK), :], vin, sem)

        def wait_read(vin, sem):
            pltpu.make_async_copy(
                tab_hbm.at[pl.ds(0, 4 * P_CHUNK), :], vin, sem).wait()

        def drain_write(vout, sem):
            pltpu.make_async_copy(vout, tabp_hbm.at[pl.ds(0, P_CHUNK)],
                                  sem).wait()

        def shuffle_write(g, vin, vout, sem):
            for p in range(P_CHUNK):
                for q in range(4):
                    for j in range(2):
                        vout[p, pl.ds(q * OUT_DIM + j * LANES, LANES)] = (
                            vin[4 * p + q, pl.ds(j * LANES, LANES)]
                        )
            pltpu.async_copy(vout, tabp_hbm.at[pl.ds(p_at(g), P_CHUNK)], sem)

        start_read(0, vin_a, sem_ra)
        start_read(1, vin_b, sem_rb)
        start_read(2, vin_c, sem_rc)

        bufs = ((vin_a, sem_ra, vout_a, sem_wa),
                (vin_b, sem_rb, vout_b, sem_wb),
                (vin_c, sem_rc, vout_c, sem_wc))

        def triple_body(t, _):
            g = 3 * t
            for i, (vin, sem_r, vout, sem_w) in enumerate(bufs):
                wait_read(vin, sem_r)

                @pl.when(t > 0)
                def _():
                    drain_write(vout, sem_w)

                shuffle_write(g + i, vin, vout, sem_w)

                @pl.when(g + i + 3 < P_CHUNKS)
                def _():
                    start_read(g + i + 3, vin, sem_r)
            return 0

        lax.fori_loop(0, P_CHUNKS // 3, triple_body, 0)
        for _, _, vout, sem_w in bufs:
            drain_write(vout, sem_w)

    return k1


def _build_lookup(num_rows):
    rows_per_w = num_rows // NW
    chunks = rows_per_w // ROWS_PER_CHUNK
    assert chunks % 2 == 1

    @functools.partial(
        pl.kernel,
        mesh=_mesh(),
        compiler_params=pltpu.CompilerParams(use_tc_tiling_on_sc=False),
        out_type=jax.ShapeDtypeStruct((num_rows * OUT_DIM,), jnp.float32),
        scratch_types=[
            pltpu.VMEM((LOOKUPS,), jnp.int32),
            pltpu.VMEM((LOOKUPS,), jnp.int32),
            pltpu.VMEM((LOOKUPS,), jnp.int32),
            pltpu.VMEM((LOOKUPS, OUT_DIM), jnp.float32),
            pltpu.VMEM((LOOKUPS, OUT_DIM), jnp.float32),
            pltpu.VMEM((ROWS_PER_CHUNK * OUT_DIM,), jnp.float32),
            pltpu.SemaphoreType.DMA,
            pltpu.SemaphoreType.DMA,
        ],
    )
    def k2(tab_hbm, idx_hbm, offs_hbm, out_hbm,
           idx_a, idx_b, offs_v, rows_a, rows_b, out_v, sem_a, sem_b):
        wid = lax.axis_index("s") * NC + lax.axis_index("c")
        pltpu.sync_copy(offs_hbm, offs_v)
        idx_base = wid * (rows_per_w * INPUT_DIM)

        def stage(g, idx_v, rows_v, sem):
            off = pl.multiple_of(idx_base + g * LOOKUPS, 8)
            pltpu.sync_copy(idx_hbm.at[pl.ds(off, LOOKUPS)], idx_v)

            def offs_body(j, _):
                for kk in range(128 // LANES):
                    sl = pl.ds(j * 128 + kk * LANES, LANES)
                    idx_v[sl] = idx_v[sl] + offs_v[sl]
                return 0

            lax.fori_loop(0, LOOKUPS // 128, offs_body, 0)
            return pltpu.async_copy(tab_hbm.at[idx_v], rows_v, sem)

        def process(g, rows_v):
            def acc_body(c, _):
                base = c * INPUT_DIM
                a0 = rows_v[base, pl.ds(0, LANES)]
                a1 = rows_v[base, pl.ds(LANES, LANES)]
                for i in range(1, INPUT_DIM):
                    a0 = a0 + rows_v[base + i, pl.ds(0, LANES)]
                    a1 = a1 + rows_v[base + i, pl.ds(LANES, LANES)]
                out_v[pl.ds(c * OUT_DIM, LANES)] = a0
                out_v[pl.ds(c * OUT_DIM + LANES, LANES)] = a1
                return 0

            lax.fori_loop(0, ROWS_PER_CHUNK, acc_body, 0)
            off = pl.multiple_of(
                (wid * rows_per_w + g * ROWS_PER_CHUNK) * OUT_DIM, 8)
            pltpu.sync_copy(
                out_v, out_hbm.at[pl.ds(off, ROWS_PER_CHUNK * OUT_DIM)]
            )

        stage(0, idx_a, rows_a, sem_a)

        def pair_body(t, _):
            g = 2 * t
            cp_b = stage(g + 1, idx_b, rows_b, sem_b)
            pltpu.make_async_copy(tab_hbm.at[idx_a], rows_a, sem_a).wait()
            process(g, rows_a)
            stage(g + 2, idx_a, rows_a, sem_a)
            cp_b.wait()
            process(g + 1, rows_b)
            return 0

        lax.fori_loop(0, chunks // 2, pair_body, 0)
        pltpu.make_async_copy(tab_hbm.at[idx_a], rows_a, sem_a).wait()
        process(chunks - 1, rows_a)

    return k2


def kernel(int_vec, tables):
    bs, num_nodes, input_dim = int_vec.shape
    num_rows = bs * num_nodes
    tab2d = tables.reshape(input_dim * tables.shape[1], tables.shape[2])
    tabp = _build_depad()(tab2d)
    tab_sc = tabp.reshape(input_dim * tables.shape[1], tables.shape[2])
    idx_flat = int_vec.reshape(num_rows * input_dim)
    offs = jnp.tile(
        jnp.arange(INPUT_DIM, dtype=jnp.int32) * NUM_EMB, ROWS_PER_CHUNK
    )
    out = _build_lookup(num_rows)(tab_sc, idx_flat, offs)
    return out.reshape(bs, num_nodes, tables.shape[2])
